# baseline (speedup 1.0000x reference)
import functools

import jax
import jax.numpy as jnp
from jax import lax
from jax.experimental import pallas as pl
from jax.experimental.pallas import tpu as pltpu
from jax.experimental.pallas import tpu_sc as plsc

_N = 4096
_C = 32
_NC = 2
_NS = 16
_NW = _NC * _NS
_BPW = _N // _NW
_CH = 8


def _table_body(z_ref, tab_ref):
    z = z_ref[0, :]
    c = lax.broadcasted_iota(jnp.int32, (_C, 1), 0)
    tab_ref[...] = (z[None, :] == c).astype(jnp.float32)


def _build_table(z2):
    return pl.pallas_call(
        _table_body,
        out_shape=jax.ShapeDtypeStruct((_C, _N), jnp.float32),
    )(z2)


def _sc_body(tab_hbm, z_hbm, out_hbm, idx_v, tab_v, sem, tsem):
    sid = lax.axis_index("s")
    wid = sid * _NC + lax.axis_index("c")
    base = wid * _BPW
    pltpu.sync_copy(z_hbm.at[pl.ds(base, _BPW)], idx_v)
    cbase = (sid // 2) * _C
    @pl.when(sid % 2 == 0)
    def _():
        pltpu.sync_copy(tab_hbm, tab_v.at[pl.ds(cbase, _C)])
    plsc.subcore_barrier()
    lanes = lax.iota(jnp.int32, 16)
    for i in range(_BPW):
        if i % 16 == 0:
            vec = idx_v[pl.ds(i, 16)]
        zi = jnp.squeeze(lax.slice(vec, (i % 16,), (i % 16 + 1,)))
        pltpu.async_copy(
            tab_v.at[pl.ds(cbase + zi, 1)], out_hbm.at[pl.ds(base + i, 1)], sem)
    pltpu.make_async_copy(
        out_hbm.at[pl.ds(base, _BPW)], out_hbm.at[pl.ds(base, _BPW)], sem
    ).wait()


_sc_gather = functools.partial(
    pl.kernel,
    out_type=jax.ShapeDtypeStruct((_N, _N), jnp.float32),
    mesh=plsc.VectorSubcoreMesh(core_axis_name="c", subcore_axis_name="s"),
    scratch_types=[
        pltpu.VMEM((_BPW,), jnp.int32),
        pltpu.VMEM_SHARED((8 * _C, _N), jnp.float32),
        pltpu.SemaphoreType.DMA,
        pltpu.SemaphoreType.DMA,
    ],
)(_sc_body)


def kernel(Z):
    z = Z.reshape(-1).astype(jnp.int32)
    tab = _build_table(z.reshape(1, _N))
    return _sc_gather(tab, z)


# speedup vs baseline: 1.1658x; 1.1658x over previous
import functools

import jax
import jax.numpy as jnp
from jax import lax
from jax.experimental import pallas as pl
from jax.experimental.pallas import tpu as pltpu
from jax.experimental.pallas import tpu_sc as plsc

_N = 4096
_C = 32
_NC = 2
_NS = 16
_NW = _NC * _NS
_BPW = _N // _NW
_G = 8


def _table_body(z_ref, tab_ref):
    z = z_ref[0, :]
    c = lax.broadcasted_iota(jnp.int32, (_C, 1), 0)
    tab_ref[...] = (z[None, :] == c).astype(jnp.float32)


def _build_table(z2):
    return pl.pallas_call(
        _table_body,
        out_shape=jax.ShapeDtypeStruct((_C, _N), jnp.float32),
    )(z2)


def _sc_body(tab_hbm, z_hbm, out_hbm, idx_v, tab_v, st0, st1, sem, ws0, ws1):
    sid = lax.axis_index("s")
    wid = sid * _NC + lax.axis_index("c")
    base = wid * _BPW
    pltpu.sync_copy(z_hbm.at[pl.ds(base, _BPW)], idx_v)
    @pl.when(sid == 0)
    def _():
        pltpu.sync_copy(tab_hbm, tab_v)
    plsc.subcore_barrier()
    sts = (st0, st1)
    wsems = (ws0, ws1)
    n_groups = _BPW // _G
    wcp = [None] * n_groups
    for g in range(n_groups):
        st = sts[g % 2]
        if g >= 2:
            wcp[g - 2].wait()
        for j in range(_G):
            i = g * _G + j
            if i % 16 == 0:
                vec = idx_v[pl.ds(i, 16)]
            zi = jnp.squeeze(lax.slice(vec, (i % 16,), (i % 16 + 1,)))
            pltpu.async_copy(
                tab_v.at[pl.ds(zi, 1)], st.at[pl.ds(j, 1)], sem)
        pltpu.make_async_copy(tab_v.at[pl.ds(0, _G)], st, sem).wait()
        wcp[g] = pltpu.async_copy(
            st, out_hbm.at[pl.ds(base + g * _G, _G)], wsems[g % 2])
    wcp[n_groups - 2].wait()
    wcp[n_groups - 1].wait()


_sc_gather = functools.partial(
    pl.kernel,
    out_type=jax.ShapeDtypeStruct((_N, _N), jnp.float32),
    mesh=plsc.VectorSubcoreMesh(core_axis_name="c", subcore_axis_name="s"),
    scratch_types=[
        pltpu.VMEM((_BPW,), jnp.int32),
        pltpu.VMEM_SHARED((_C, _N), jnp.float32),
        pltpu.VMEM((_G, _N), jnp.float32),
        pltpu.VMEM((_G, _N), jnp.float32),
        pltpu.SemaphoreType.DMA,
        pltpu.SemaphoreType.DMA,
        pltpu.SemaphoreType.DMA,
    ],
)(_sc_body)


def kernel(Z):
    z = Z.reshape(-1).astype(jnp.int32)
    tab = _build_table(z.reshape(1, _N))
    return _sc_gather(tab, z)
